# baseline (speedup 1.0000x reference)
import jax
import jax.numpy as jnp
from jax import lax
from jax.experimental import pallas as pl
from jax.experimental.pallas import tpu as pltpu
from jax.experimental.pallas import tpu_sc as plsc

COUNT_DIM = 100
N_EMBD = 64
BATCH = 16384
NVALS = 100

L = 16
NC = 2
NS = 16
NW = NC * NS
CB = 256
NCHUNK = BATCH // CB
ITEMS = COUNT_DIM * NCHUNK
IPW = ITEMS // NW

_DNUMS = lax.GatherDimensionNumbers(
    offset_dims=(), collapsed_slice_dims=(0,), start_index_map=(0,))


def _lane_bcast(vec, e):
    idx = jnp.full((L, 1), e, jnp.int32)
    return lax.gather(vec, idx, _DNUMS, (1,),
                      mode=lax.GatherScatterMode.PROMISE_IN_BOUNDS)


def _sc_body(cntT_hbm, val_hbm, bit_hbm, out_hbm,
             val_v, bit_v, cnt_v, ob0, ob1):
    wid = lax.axis_index("s") * NC + lax.axis_index("c")

    pltpu.sync_copy(val_hbm, val_v)
    pltpu.sync_copy(bit_hbm, bit_v)

    col0 = lax.iota(jnp.int32, L)

    def do_item(t, ob):
        item = wid * IPW + t
        d = item // NCHUNK
        ch = item - d * NCHUNK
        b0 = ch * CB

        pltpu.sync_copy(cntT_hbm.at[d, pl.ds(b0, CB)], cnt_v)

        bits = [bit_v[pl.ds(d * N_EMBD + L * j, L)] for j in range(4)]

        @plsc.parallel_loop(0, CB // L, unroll=2)
        def group_body(g):
            cvec = cnt_v[pl.ds(g * L, L)]
            for e in range(L):
                base = _lane_bcast(cvec, e) * N_EMBD
                vals = [plsc.load_gather(val_v, [base + (16 * j) + col0])
                        for j in range(4)]
                row = g * L + e
                for j in range(4):
                    ob[row, pl.ds(16 * j, L)] = vals[j] + bits[j]

        pltpu.sync_copy(ob, out_hbm.at[pl.ds(b0, CB), d])

    def pair_body(t2, carry):
        do_item(2 * t2, ob0)
        do_item(2 * t2 + 1, ob1)
        return carry

    lax.fori_loop(0, IPW // 2, pair_body, 0, unroll=False)


def kernel(count, val_emb, bit_emb):
    cnt_t = count.astype(jnp.int32).T
    val_flat = val_emb.reshape(-1)
    bit_flat = bit_emb.reshape(-1)

    mesh = plsc.VectorSubcoreMesh(core_axis_name="c", subcore_axis_name="s")
    f = pl.kernel(
        _sc_body,
        mesh=mesh,
        compiler_params=pltpu.CompilerParams(needs_layout_passes=False),
        out_type=jax.ShapeDtypeStruct((BATCH, COUNT_DIM, N_EMBD), jnp.float32),
        scratch_types=[
            pltpu.VMEM((NVALS * N_EMBD,), jnp.float32),
            pltpu.VMEM((COUNT_DIM * N_EMBD,), jnp.float32),
            pltpu.VMEM((CB,), jnp.int32),
            pltpu.VMEM((CB, N_EMBD), jnp.float32),
            pltpu.VMEM((CB, N_EMBD), jnp.float32),
        ],
    )
    return f(cnt_t, val_flat, bit_flat)
